# Initial kernel scaffold; baseline (speedup 1.0000x reference)
import jax
import jax.numpy as jnp
from jax.experimental import pallas as pl


def kernel(inputs, kernel):
    raise NotImplementedError("write your pallas kernel here")


# speedup vs baseline: 7.3682x; 7.3682x over previous
import functools

import jax
import jax.numpy as jnp
from jax.experimental import pallas as pl

_MAX_LENGTH = 200


def _body(table_ref, out_ref, *, nrows):
    tb, seq, dim = out_ref.shape
    vp = table_ref.shape[0]
    pos = jax.lax.broadcasted_iota(jnp.int32, (seq, vp), 0)
    pos = jnp.maximum(pos, _MAX_LENGTH)
    pos = jnp.minimum(pos, nrows - 1)
    col = jax.lax.broadcasted_iota(jnp.int32, (seq, vp), 1)
    onehot = (col == pos).astype(table_ref.dtype)
    gathered = jnp.dot(onehot, table_ref[...], preferred_element_type=jnp.float32)
    out_ref[...] = jnp.broadcast_to(gathered[None], (tb, seq, dim))


def kernel(inputs, kernel):
    batch, seq = inputs.shape
    nrows, dim = kernel.shape
    vp = (nrows + 7) // 8 * 8
    table = jnp.zeros((vp, dim), kernel.dtype).at[:nrows].set(kernel)
    tb = 128
    out = pl.pallas_call(
        functools.partial(_body, nrows=nrows),
        grid=(batch // tb,),
        in_specs=[pl.BlockSpec((vp, dim), lambda i: (0, 0))],
        out_specs=pl.BlockSpec((tb, seq, dim), lambda i: (i, 0, 0)),
        out_shape=jax.ShapeDtypeStruct((batch, seq, dim), jnp.float32),
    )(table)
    return out


# speedup vs baseline: 12.0061x; 1.6294x over previous
import functools

import jax
import jax.numpy as jnp
from jax.experimental import pallas as pl

_MAX_LENGTH = 200


def _gather_body(table_ref, out_ref, *, nrows):
    seq, dim = out_ref.shape
    vp = table_ref.shape[0]
    pos = jax.lax.broadcasted_iota(jnp.int32, (seq, vp), 0)
    pos = jnp.maximum(pos, _MAX_LENGTH)
    pos = jnp.minimum(pos, nrows - 1)
    col = jax.lax.broadcasted_iota(jnp.int32, (seq, vp), 1)
    onehot = (col == pos).astype(table_ref.dtype)
    out_ref[...] = jnp.dot(onehot, table_ref[...],
                           preferred_element_type=jnp.float32)


def _bcast_body(row_ref, out_ref):
    out_ref[...] = jnp.broadcast_to(row_ref[...], out_ref.shape)


def kernel(inputs, kernel):
    batch, seq = inputs.shape
    nrows, dim = kernel.shape
    vp = (nrows + 7) // 8 * 8
    table = jnp.zeros((vp, dim), kernel.dtype).at[:nrows].set(kernel)

    gathered = pl.pallas_call(
        functools.partial(_gather_body, nrows=nrows),
        out_shape=jax.ShapeDtypeStruct((seq, dim), jnp.float32),
    )(table)

    row = gathered.reshape(1, seq * dim)
    tb = 128
    out = pl.pallas_call(
        _bcast_body,
        grid=(batch // tb,),
        in_specs=[pl.BlockSpec((1, seq * dim), lambda i: (0, 0))],
        out_specs=pl.BlockSpec((tb, seq * dim), lambda i: (i, 0)),
        out_shape=jax.ShapeDtypeStruct((batch, seq * dim), jnp.float32),
    )(row)
    return out.reshape(batch, seq, dim)
